# Initial kernel scaffold; baseline (speedup 1.0000x reference)
import jax
import jax.numpy as jnp
from jax.experimental import pallas as pl


def kernel(pos, cell, Sij, W_embed, W_x, W_r1, W_r2, W_r3, W_up, W_out, W_sc, W_prod, W_read, edge_index, num_edges, batch, ptr, species):
    raise NotImplementedError("write your pallas kernel here")


# speedup vs baseline: 5.6429x; 5.6429x over previous
import jax
import jax.numpy as jnp
import numpy as np
from jax.experimental import pallas as pl

_CUTOFF = 6.0
_AVG = 40.0
_C = 32
_K = 16
_EB = 2000
_NB = 500
_NRB = 2000


def _mm(a, b):
    return jnp.dot(a, b, precision=jax.lax.Precision.HIGHEST)


def _silu(x):
    return x * jax.nn.sigmoid(x)


def _edge_embed_kernel(ps, pd, sij, cellr, wr1, wr2, wr3, y_out, w_out):
    cell = cellr[0]
    shift = _mm(sij[...], cell)
    rij = (pd[...] - ps[...] + shift) * (1.0 / _CUTOFF)
    l2 = jnp.sum(rij * rij, axis=1, keepdims=True)
    lengths = jnp.sqrt(l2)
    mask = (lengths > 0.0).astype(jnp.float32)
    xl = jnp.clip(lengths, 1e-9, None)
    u = rij / xl
    x = u[:, 0:1]
    y = u[:, 1:2]
    z = u[:, 2:3]
    s3 = float(np.sqrt(3.0)); s15 = float(np.sqrt(15.0)); s5 = float(np.sqrt(5.0))
    s70 = float(np.sqrt(70.0)); s105 = float(np.sqrt(105.0))
    s42 = float(np.sqrt(42.0)); s7 = float(np.sqrt(7.0))
    x2 = x * x; y2 = y * y; z2 = z * z
    cols = [
        jnp.ones_like(x),
        s3 * x, s3 * y, s3 * z,
        s15 * x * y, s15 * y * z, (s5 / 2.0) * (3.0 * z2 - 1.0),
        s15 * x * z, (s15 / 2.0) * (x2 - y2),
        (s70 / 4.0) * y * (3.0 * x2 - y2), s105 * x * y * z,
        (s42 / 4.0) * y * (5.0 * z2 - 1.0), (s7 / 2.0) * z * (5.0 * z2 - 3.0),
        (s42 / 4.0) * x * (5.0 * z2 - 1.0), (s105 / 2.0) * z * (x2 - y2),
        (s70 / 4.0) * x * (x2 - 3.0 * y2),
    ]
    y_out[...] = jnp.concatenate(cols, axis=1)
    n = jax.lax.broadcasted_iota(jnp.int32, (1, 8), 1).astype(jnp.float32) + 1.0
    b = float(np.sqrt(2.0)) * jnp.sin(n * np.pi * xl) / xl
    env = 1.0 - 6.0 * xl ** 2 + 8.0 * xl ** 3 - 3.0 * xl ** 4
    env = env * (xl < 1.0).astype(jnp.float32)
    ef = b * env
    scale = mask * (1.0 / _AVG)
    ws = []
    for l in range(3):
        a = _silu(_mm(ef, wr1[l]))
        a = _silu(_mm(a, wr2[l]))
        ws.append(_mm(a, wr3[l]) * scale)
    w_out[...] = jnp.concatenate(ws, axis=1)


def _make_msg0_kernel(lidx):
    def _k(m0, yy, wef, out):
        wl = wef[:, lidx * _C:(lidx + 1) * _C]
        base = wl * m0[...]
        for k in range(_K):
            yk = yy[:, k:k + 1]
            if k == 0:
                out[:, 0:_C] = 2.0 * base
            else:
                out[:, k * _C:(k + 1) * _C] = base * yk
    return _k


def _make_msg_kernel(lidx):
    def _k(m, yy, wef, out):
        wl = wef[:, lidx * _C:(lidx + 1) * _C]
        m0 = m[:, 0:_C]
        for k in range(_K):
            yk = yy[:, k:k + 1]
            out[:, k * _C:(k + 1) * _C] = wl * (m[:, k * _C:(k + 1) * _C] + m0 * yk)
    return _k


def _node_mid_kernel(agg, hf, xnf, spm, wout, wprod, wsc, wupn, h_out, hu_out):
    hn = _mm(agg[...], wout[...])
    p = _mm(hn * xnf[...], wprod[...])
    h = hf[...]
    sc = spm[:, 0:1] * _mm(h, wsc[0])
    for s in range(1, 4):
        sc = sc + spm[:, s:s + 1] * _mm(h, wsc[s])
    hx = p + sc
    h_out[...] = hx
    hu_out[...] = _mm(hx, wupn[...])


def _node_last_kernel(agg, hf, xnf, spm, wout, wprod, wsc, h_out):
    hn = _mm(agg[...], wout[...])
    p = _mm(hn * xnf[...], wprod[...])
    h = hf[...]
    sc = spm[:, 0:1] * _mm(h, wsc[0])
    for s in range(1, 4):
        sc = sc + spm[:, s:s + 1] * _mm(h, wsc[s])
    h_out[...] = p + sc


def _energy_kernel(h1, h2, h3, a0, a1, a2, out):
    acc = _silu(_mm(h1[0, :, _C:4 * _C], a0[...]))
    acc = acc + _silu(_mm(h2[0, :, _C:4 * _C], a1[...]))
    acc = acc + _silu(_mm(h3[0, :, _C:4 * _C], a2[...]))
    out[0, 0, :] = jnp.sum(acc, axis=0)


def _row_spec(b, w):
    return pl.BlockSpec((b, w), lambda i: (i, 0))


def _full_spec(shape):
    nd = len(shape)
    return pl.BlockSpec(shape, lambda i: (0,) * nd)


def kernel(pos, cell, Sij, W_embed, W_x, W_r1, W_r2, W_r3, W_up, W_out, W_sc, W_prod, W_read, edge_index, num_edges, batch, ptr, species):
    N = pos.shape[0]
    E = edge_index.shape[1]
    G = ptr.shape[0] - 1
    C, K = _C, _K
    snd, rcv = edge_index[0], edge_index[1]
    eg = E // (G * _EB)

    pos_s = pos[snd]
    pos_d = pos[rcv]
    cellr = cell

    Y, Wef = pl.pallas_call(
        _edge_embed_kernel,
        grid=(E // _EB,),
        in_specs=[
            _row_spec(_EB, 3), _row_spec(_EB, 3), _row_spec(_EB, 3),
            pl.BlockSpec((1, 3, 3), lambda i: (i // eg, 0, 0)),
            _full_spec((3, 8, 64)), _full_spec((3, 64, 64)), _full_spec((3, 64, C)),
        ],
        out_specs=[_row_spec(_EB, K), _row_spec(_EB, 3 * C)],
        out_shape=[
            jax.ShapeDtypeStruct((E, K), jnp.float32),
            jax.ShapeDtypeStruct((E, 3 * C), jnp.float32),
        ],
    )(pos_s, pos_d, Sij, cellr, W_r1, W_r2, W_r3)

    sp1h = jax.nn.one_hot(species, 4, dtype=jnp.float32)
    nf0 = sp1h @ W_embed
    xnf = nf0 @ W_x
    xnf_exp = jnp.repeat(xnf, K, axis=0)
    spm_exp = jnp.repeat(sp1h, K, axis=0)
    h0f = jnp.zeros((N, K, C), jnp.float32).at[:, 0, :].set(nf0).reshape(N * K, C)
    hu0_k0 = nf0 @ W_up[0]

    eye3 = jnp.eye(3, dtype=jnp.float32)
    Amats = [jnp.einsum('cm,vw->vcmw', W_read[l], eye3).reshape(3 * C, 9) for l in range(3)]

    def msg_call(body, m_arr, mw):
        return pl.pallas_call(
            body,
            grid=(E // _EB,),
            in_specs=[_row_spec(_EB, mw), _row_spec(_EB, K), _row_spec(_EB, 3 * C)],
            out_specs=_row_spec(_EB, K * C),
            out_shape=jax.ShapeDtypeStruct((E, K * C), jnp.float32),
        )(m_arr, Y, Wef)

    def node_mid(agg_f, h_f, l):
        return pl.pallas_call(
            _node_mid_kernel,
            grid=(N * K // _NRB,),
            in_specs=[
                _row_spec(_NRB, C), _row_spec(_NRB, C), _row_spec(_NRB, C),
                _row_spec(_NRB, 4),
                _full_spec((C, C)), _full_spec((C, C)), _full_spec((4, C, C)), _full_spec((C, C)),
            ],
            out_specs=[_row_spec(_NRB, C), _row_spec(_NRB, C)],
            out_shape=[
                jax.ShapeDtypeStruct((N * K, C), jnp.float32),
                jax.ShapeDtypeStruct((N * K, C), jnp.float32),
            ],
        )(agg_f, h_f, xnf_exp, spm_exp, W_out[l], W_prod[l], W_sc[l], W_up[l + 1])

    m0g = hu0_k0[snd]
    msg = msg_call(_make_msg0_kernel(0), m0g, C)
    aggf = jnp.zeros((N, K * C), jnp.float32).at[rcv].add(msg).reshape(N * K, C)
    h1f, hu1f = node_mid(aggf, h0f, 0)

    m = hu1f.reshape(N, K * C)[snd]
    msg = msg_call(_make_msg_kernel(1), m, K * C)
    aggf = jnp.zeros((N, K * C), jnp.float32).at[rcv].add(msg).reshape(N * K, C)
    h2f, hu2f = node_mid(aggf, h1f, 1)

    m = hu2f.reshape(N, K * C)[snd]
    msg = msg_call(_make_msg_kernel(2), m, K * C)
    aggf = jnp.zeros((N, K * C), jnp.float32).at[rcv].add(msg).reshape(N * K, C)
    h3f = pl.pallas_call(
        _node_last_kernel,
        grid=(N * K // _NRB,),
        in_specs=[
            _row_spec(_NRB, C), _row_spec(_NRB, C), _row_spec(_NRB, C),
            _row_spec(_NRB, 4),
            _full_spec((C, C)), _full_spec((C, C)), _full_spec((4, C, C)),
        ],
        out_specs=_row_spec(_NRB, C),
        out_shape=jax.ShapeDtypeStruct((N * K, C), jnp.float32),
    )(aggf, h2f, xnf_exp, spm_exp, W_out[2], W_prod[2], W_sc[2])

    h1 = h1f.reshape(G, _NB, K * C)
    h2 = h2f.reshape(G, _NB, K * C)
    h3 = h3f.reshape(G, _NB, K * C)
    g_spec = pl.BlockSpec((1, _NB, K * C), lambda i: (i, 0, 0))
    eng = pl.pallas_call(
        _energy_kernel,
        grid=(G,),
        in_specs=[
            g_spec, g_spec, g_spec,
            _full_spec((3 * C, 9)), _full_spec((3 * C, 9)), _full_spec((3 * C, 9)),
        ],
        out_specs=pl.BlockSpec((1, 1, 9), lambda i: (i, 0, 0)),
        out_shape=jax.ShapeDtypeStruct((G, 1, 9), jnp.float32),
    )(h1, h2, h3, Amats[0], Amats[1], Amats[2])
    return eng.reshape(G, 3, 3)
